# baseline (speedup 1.0000x reference)
import jax
import jax.numpy as jnp
from jax import lax
from jax.experimental import pallas as pl
from jax.experimental.pallas import tpu as pltpu
from jax.experimental.pallas import tpu_sc as plsc

B = 10
N_PAD = 1000
N_PADDED = 1024
E_PAD = 16000
NODE_DIM = 256
EDGE_DIM = 16
HIDDEN = 256
LEAKY = 0.01
BN_EPS = 1e-5

TOTAL_N = B * N_PADDED
VALID_N = B * N_PAD
TOTAL_E = B * E_PAD
FB = 16
E_CHUNK = 640
N_CHUNKS = E_PAD // E_CHUNK
N_GROUPS = E_CHUNK // 16


def _leaky(v):
    return jnp.where(v > 0, v, LEAKY * v)


def _valid_mask(shape, col0):
    col = col0 + lax.broadcasted_iota(jnp.int32, shape, 1)
    return (col % N_PADDED) < N_PAD


def _pre_h_kernel(x_ref, w_ref, b_ref, o_ref):
    z = (
        lax.dot_general(w_ref[...], x_ref[0], (((1,), (1,)), ((), ())),
                        preferred_element_type=jnp.float32)
        + b_ref[...]
    )
    o_ref[:, : N_PAD] = z
    o_ref[:, N_PAD:] = jnp.zeros((HIDDEN, N_PADDED - N_PAD), jnp.float32)


def _pre_h(x, w, b):
    return pl.pallas_call(
        _pre_h_kernel,
        grid=(B,),
        in_specs=[
            pl.BlockSpec((1, N_PAD, NODE_DIM), lambda g: (g, 0, 0)),
            pl.BlockSpec((HIDDEN, NODE_DIM), lambda g: (0, 0)),
            pl.BlockSpec((HIDDEN, 1), lambda g: (0, 0)),
        ],
        out_specs=pl.BlockSpec((HIDDEN, N_PADDED), lambda g: (0, g)),
        out_shape=jax.ShapeDtypeStruct((HIDDEN, TOTAL_N), jnp.float32),
    )(x, w, b.reshape(-1, 1))


def _pre_e_kernel(ef_ref, w_ref, b_ref, o_ref):
    o_ref[...] = (
        lax.dot_general(w_ref[...], ef_ref[...], (((1,), (1,)), ((), ())),
                        preferred_element_type=jnp.float32)
        + b_ref[...]
    )


def _pre_e(ef, w, b, col_blk=6400):
    n = ef.shape[0]
    return pl.pallas_call(
        _pre_e_kernel,
        grid=(n // col_blk,),
        in_specs=[
            pl.BlockSpec((col_blk, EDGE_DIM), lambda i: (i, 0)),
            pl.BlockSpec((HIDDEN, EDGE_DIM), lambda i: (0, 0)),
            pl.BlockSpec((HIDDEN, 1), lambda i: (0, 0)),
        ],
        out_specs=pl.BlockSpec((HIDDEN, col_blk), lambda i: (0, i)),
        out_shape=jax.ShapeDtypeStruct((HIDDEN, n), jnp.float32),
    )(ef, w, b.reshape(-1, 1))


def _layer_mm_kernel(h_ref, agg_ref, w1_ref, b1_ref, w2_ref, b2_ref,
                     z_ref, s_ref, q_ref):
    u = h_ref[...] + agg_ref[...]
    z1 = _leaky(
        jnp.dot(w1_ref[...], u, preferred_element_type=jnp.float32)
        + b1_ref[...]
    )
    z2 = (
        jnp.dot(w2_ref[...], z1, preferred_element_type=jnp.float32)
        + b2_ref[...]
    )
    z_ref[...] = z2

    @pl.when(pl.program_id(0) == 0)
    def _():
        s_ref[...] = jnp.zeros_like(s_ref)
        q_ref[...] = jnp.zeros_like(q_ref)

    col_blk = z2.shape[1]
    mask = _valid_mask(z2.shape, pl.program_id(0) * col_blk)
    zm = jnp.where(mask, z2, 0.0)
    s_ref[...] += jnp.sum(zm, axis=1, keepdims=True)
    q_ref[...] += jnp.sum(zm * z2, axis=1, keepdims=True)


def _layer_mm(hT, aggT, w1, b1, w2, b2, col_blk=2048):
    m, n = hT.shape
    grid = (n // col_blk,)
    return pl.pallas_call(
        _layer_mm_kernel,
        grid=grid,
        in_specs=[
            pl.BlockSpec((m, col_blk), lambda i: (0, i)),
            pl.BlockSpec((m, col_blk), lambda i: (0, i)),
            pl.BlockSpec((HIDDEN, m), lambda i: (0, 0)),
            pl.BlockSpec((HIDDEN, 1), lambda i: (0, 0)),
            pl.BlockSpec((HIDDEN, HIDDEN), lambda i: (0, 0)),
            pl.BlockSpec((HIDDEN, 1), lambda i: (0, 0)),
        ],
        out_specs=[
            pl.BlockSpec((HIDDEN, col_blk), lambda i: (0, i)),
            pl.BlockSpec((HIDDEN, 1), lambda i: (0, 0)),
            pl.BlockSpec((HIDDEN, 1), lambda i: (0, 0)),
        ],
        out_shape=[
            jax.ShapeDtypeStruct((HIDDEN, n), jnp.float32),
            jax.ShapeDtypeStruct((HIDDEN, 1), jnp.float32),
            jax.ShapeDtypeStruct((HIDDEN, 1), jnp.float32),
        ],
    )(hT, aggT, w1, b1.reshape(-1, 1), w2, b2.reshape(-1, 1))


def _bn_kernel(z_ref, s_ref, q_ref, g_ref, be_ref, o_ref):
    inv_n = 1.0 / VALID_N
    mean = s_ref[...] * inv_n
    var = q_ref[...] * inv_n - mean * mean
    inv = lax.rsqrt(var + BN_EPS)
    o_ref[...] = _leaky((z_ref[...] - mean) * inv * g_ref[...] + be_ref[...])


def _bn_apply(zT, s, q, gamma, beta, col_blk=2048):
    m, n = zT.shape
    grid = (n // col_blk,)
    return pl.pallas_call(
        _bn_kernel,
        grid=grid,
        in_specs=[
            pl.BlockSpec((m, col_blk), lambda i: (0, i)),
            pl.BlockSpec((m, 1), lambda i: (0, 0)),
            pl.BlockSpec((m, 1), lambda i: (0, 0)),
            pl.BlockSpec((m, 1), lambda i: (0, 0)),
            pl.BlockSpec((m, 1), lambda i: (0, 0)),
        ],
        out_specs=pl.BlockSpec((m, col_blk), lambda i: (0, i)),
        out_shape=jax.ShapeDtypeStruct((m, n), jnp.float32),
    )(zT, s, q, gamma.reshape(-1, 1), beta.reshape(-1, 1))


def _pool_bn_kernel(z_ref, s_ref, q_ref, g_ref, be_ref, o_ref):
    inv_n = 1.0 / VALID_N
    mean = s_ref[...] * inv_n
    var = q_ref[...] * inv_n - mean * mean
    inv = lax.rsqrt(var + BN_EPS)
    h = _leaky((z_ref[...] - mean) * inv * g_ref[...] + be_ref[...])
    hm = jnp.where(_valid_mask(h.shape, 0), h, 0.0)
    p = jnp.sum(hm.reshape(HIDDEN, B, N_PADDED), axis=2) * (1.0 / N_PAD)
    ss = jnp.sum(p * p, axis=0, keepdims=True)
    pn = p / jnp.maximum(jnp.sqrt(ss), 1e-12)
    o_ref[...] = pn.T


def _pool_bn(zT, st, q, gamma, beta):
    return pl.pallas_call(
        _pool_bn_kernel,
        out_shape=jax.ShapeDtypeStruct((B, HIDDEN), jnp.float32),
    )(zT, st, q, gamma.reshape(-1, 1), beta.reshape(-1, 1))


_GATHER_DNUMS = lax.GatherDimensionNumbers(
    offset_dims=(), collapsed_slice_dims=(0,), start_index_map=(0,))


def _lane_pick(vec, idx):
    return lax.gather(vec, idx[:, None], _GATHER_DNUMS, slice_sizes=(1,),
                      mode=lax.GatherScatterMode.PROMISE_IN_BOUNDS)


def _sc_gine_body(h_hbm, e_hbm, ei_hbm, out_hbm,
                  h_v, agg_v, e_v0, src_v0, dst_v0, e_v1, src_v1, dst_v1,
                  sem0, sem1):
    c = lax.axis_index("c")
    s = lax.axis_index("s")
    f0 = s * FB
    iota = lax.iota(jnp.int32, 16)
    zeros16 = jnp.zeros((16,), jnp.float32)
    n_steps = (B // 2) * N_CHUNKS

    def chunk_copies(t, bufs):
        e_v, src_v, dst_v, sem = bufs
        gi = t // N_CHUNKS
        ci = t % N_CHUNKS
        g = 2 * gi + c
        eoff = ci * E_CHUNK
        gbase = g * 2 * E_PAD
        return (
            pltpu.make_async_copy(
                ei_hbm.at[pl.ds(gbase + eoff, E_CHUNK)], src_v, sem),
            pltpu.make_async_copy(
                ei_hbm.at[pl.ds(gbase + E_PAD + eoff, E_CHUNK)], dst_v, sem),
            pltpu.make_async_copy(
                e_hbm.at[pl.ds(f0, FB), pl.ds(g * E_PAD + eoff, E_CHUNK)],
                e_v.at[:, pl.ds(0, E_CHUNK)], sem),
        )

    def process_chunk(bufs):
        e_v, src_v, dst_v, _ = bufs

        @plsc.parallel_loop(0, N_GROUPS, unroll=4)
        def group_body(k):
            base = k * 16
            bvec = jnp.full((16,), base, jnp.int32)
            ivec = bvec + iota
            src16 = plsc.load_gather(src_v, [ivec])
            dst16 = plsc.load_gather(dst_v, [ivec])
            for j in range(16):
                jj = jnp.full((16,), j, jnp.int32)
                sb = _lane_pick(src16, jj)
                db = _lane_pick(dst16, jj)
                pv = bvec + j
                erow = plsc.load_gather(e_v, [iota, pv])
                hrow = plsc.load_gather(h_v, [iota, sb])
                m = jnp.maximum(hrow + erow, 0.0)
                plsc.addupdate_scatter(agg_v, [iota, db], m)

    bufs0 = (e_v0, src_v0, dst_v0, sem0)
    bufs1 = (e_v1, src_v1, dst_v1, sem1)

    for cp in chunk_copies(0, bufs0):
        cp.start()

    def step_body(t, carry):
        gi = t // N_CHUNKS
        ci = t % N_CHUNKS
        g = 2 * gi + c
        n0 = g * N_PADDED

        @pl.when(ci == 0)
        def _():
            pltpu.sync_copy(h_hbm.at[pl.ds(f0, FB), pl.ds(n0, N_PADDED)],
                            h_v.at[:, pl.ds(0, N_PADDED)])

            @plsc.parallel_loop(0, N_PADDED // 8, unroll=2)
            def zero_body(i):
                base = i * 8
                for u in range(8):
                    plsc.store_scatter(
                        agg_v,
                        [iota, jnp.full((16,), base, jnp.int32) + u],
                        zeros16)

        t2 = jnp.minimum(t + 1, n_steps - 1)

        @pl.when(t % 2 == 0)
        def _():
            for cp in chunk_copies(t, bufs0):
                cp.wait()

            @pl.when(t + 1 < n_steps)
            def _():
                for cp in chunk_copies(t2, bufs1):
                    cp.start()

            process_chunk(bufs0)

        @pl.when(t % 2 == 1)
        def _():
            for cp in chunk_copies(t, bufs1):
                cp.wait()

            @pl.when(t + 1 < n_steps)
            def _():
                for cp in chunk_copies(t2, bufs0):
                    cp.start()

            process_chunk(bufs1)

        @pl.when(ci == N_CHUNKS - 1)
        def _():
            pltpu.sync_copy(agg_v.at[:, pl.ds(0, N_PADDED)],
                            out_hbm.at[pl.ds(f0, FB), pl.ds(n0, N_PADDED)])
        return carry

    lax.fori_loop(0, n_steps, step_body, 0)


def _sc_gine(hT, eT, ei_flat):
    mesh = plsc.VectorSubcoreMesh(core_axis_name="c", subcore_axis_name="s")
    return pl.kernel(
        _sc_gine_body,
        out_type=jax.ShapeDtypeStruct((HIDDEN, TOTAL_N), jnp.float32),
        mesh=mesh,
        compiler_params=pltpu.CompilerParams(
            use_tc_tiling_on_sc=False, needs_layout_passes=False),
        scratch_types=[
            pltpu.VMEM((FB, N_PADDED + 1), jnp.float32),
            pltpu.VMEM((FB, N_PADDED + 1), jnp.float32),
            pltpu.VMEM((FB, E_CHUNK + 1), jnp.float32),
            pltpu.VMEM((E_CHUNK,), jnp.int32),
            pltpu.VMEM((E_CHUNK,), jnp.int32),
            pltpu.VMEM((FB, E_CHUNK + 1), jnp.float32),
            pltpu.VMEM((E_CHUNK,), jnp.int32),
            pltpu.VMEM((E_CHUNK,), jnp.int32),
            pltpu.SemaphoreType.DMA,
            pltpu.SemaphoreType.DMA,
        ],
    )(hT, eT, ei_flat)


def kernel(x, edge_index, edge_feature, lens, params):
    del lens
    ef = edge_feature.reshape(TOTAL_E, EDGE_DIM)
    ei_flat = edge_index.reshape(B * 2 * E_PAD).astype(jnp.int32)

    hT = _pre_h(x, params["pre_mp_W"], params["pre_mp_b"])
    eT = _pre_e(ef, params["pre_edge_W"], params["pre_edge_b"])

    for l in range(2):
        lp = params["layers"][l]
        aggT = _sc_gine(hT, eT, ei_flat)
        zT, ssum, sq = _layer_mm(hT, aggT, lp["lin1_W"], lp["lin1_b"],
                                 lp["lin2_W"], lp["lin2_b"])
        if l == 0:
            hT = _bn_apply(zT, ssum, sq, lp["bn_gamma"], lp["bn_beta"])
        else:
            return _pool_bn(zT, ssum, sq, lp["bn_gamma"], lp["bn_beta"])


# speedup vs baseline: 1.1461x; 1.1461x over previous
import jax
import jax.numpy as jnp
from jax import lax
from jax.experimental import pallas as pl
from jax.experimental.pallas import tpu as pltpu
from jax.experimental.pallas import tpu_sc as plsc

B = 10
N_PAD = 1000
N_PADDED = 1024
E_PAD = 16000
NODE_DIM = 256
EDGE_DIM = 16
HIDDEN = 256
LEAKY = 0.01
BN_EPS = 1e-5

TOTAL_N = B * N_PADDED
VALID_N = B * N_PAD
TOTAL_E = B * E_PAD
FB = 16
E_CHUNK = 640
N_CHUNKS = E_PAD // E_CHUNK
N_GROUPS = E_CHUNK // 16


def _leaky(v):
    return jnp.where(v > 0, v, LEAKY * v)


def _valid_mask(shape, col0):
    col = col0 + lax.broadcasted_iota(jnp.int32, shape, 1)
    return (col % N_PADDED) < N_PAD


def _pre_h_kernel(x_ref, w_ref, b_ref, o_ref):
    z = (
        lax.dot_general(w_ref[...], x_ref[0], (((1,), (1,)), ((), ())),
                        preferred_element_type=jnp.float32)
        + b_ref[...]
    )
    o_ref[:, : N_PAD] = z
    o_ref[:, N_PAD:] = jnp.zeros((HIDDEN, N_PADDED - N_PAD), jnp.float32)


def _pre_h(x, w, b):
    return pl.pallas_call(
        _pre_h_kernel,
        grid=(B,),
        in_specs=[
            pl.BlockSpec((1, N_PAD, NODE_DIM), lambda g: (g, 0, 0)),
            pl.BlockSpec((HIDDEN, NODE_DIM), lambda g: (0, 0)),
            pl.BlockSpec((HIDDEN, 1), lambda g: (0, 0)),
        ],
        out_specs=pl.BlockSpec((HIDDEN, N_PADDED), lambda g: (0, g)),
        out_shape=jax.ShapeDtypeStruct((HIDDEN, TOTAL_N), jnp.float32),
    )(x, w, b.reshape(-1, 1))


def _pre_e_kernel(ef_ref, w_ref, b_ref, o_ref):
    o_ref[...] = (
        lax.dot_general(w_ref[...], ef_ref[...], (((1,), (1,)), ((), ())),
                        preferred_element_type=jnp.float32)
        + b_ref[...]
    )


def _pre_e(ef, w, b, col_blk=6400):
    n = ef.shape[0]
    return pl.pallas_call(
        _pre_e_kernel,
        grid=(n // col_blk,),
        in_specs=[
            pl.BlockSpec((col_blk, EDGE_DIM), lambda i: (i, 0)),
            pl.BlockSpec((HIDDEN, EDGE_DIM), lambda i: (0, 0)),
            pl.BlockSpec((HIDDEN, 1), lambda i: (0, 0)),
        ],
        out_specs=pl.BlockSpec((HIDDEN, col_blk), lambda i: (0, i)),
        out_shape=jax.ShapeDtypeStruct((HIDDEN, n), jnp.float32),
    )(ef, w, b.reshape(-1, 1))


def _layer_mm_kernel(h_ref, agg_ref, w1_ref, b1_ref, w2_ref, b2_ref,
                     z_ref, s_ref, q_ref):
    u = h_ref[...] + agg_ref[...]
    z1 = _leaky(
        jnp.dot(w1_ref[...], u, preferred_element_type=jnp.float32)
        + b1_ref[...]
    )
    z2 = (
        jnp.dot(w2_ref[...], z1, preferred_element_type=jnp.float32)
        + b2_ref[...]
    )
    z_ref[...] = z2

    @pl.when(pl.program_id(0) == 0)
    def _():
        s_ref[...] = jnp.zeros_like(s_ref)
        q_ref[...] = jnp.zeros_like(q_ref)

    col_blk = z2.shape[1]
    mask = _valid_mask(z2.shape, pl.program_id(0) * col_blk)
    zm = jnp.where(mask, z2, 0.0)
    s_ref[...] += jnp.sum(zm, axis=1, keepdims=True)
    q_ref[...] += jnp.sum(zm * z2, axis=1, keepdims=True)


def _layer_mm(hT, aggT, w1, b1, w2, b2, col_blk=2048):
    m, n = hT.shape
    grid = (n // col_blk,)
    return pl.pallas_call(
        _layer_mm_kernel,
        grid=grid,
        in_specs=[
            pl.BlockSpec((m, col_blk), lambda i: (0, i)),
            pl.BlockSpec((m, col_blk), lambda i: (0, i)),
            pl.BlockSpec((HIDDEN, m), lambda i: (0, 0)),
            pl.BlockSpec((HIDDEN, 1), lambda i: (0, 0)),
            pl.BlockSpec((HIDDEN, HIDDEN), lambda i: (0, 0)),
            pl.BlockSpec((HIDDEN, 1), lambda i: (0, 0)),
        ],
        out_specs=[
            pl.BlockSpec((HIDDEN, col_blk), lambda i: (0, i)),
            pl.BlockSpec((HIDDEN, 1), lambda i: (0, 0)),
            pl.BlockSpec((HIDDEN, 1), lambda i: (0, 0)),
        ],
        out_shape=[
            jax.ShapeDtypeStruct((HIDDEN, n), jnp.float32),
            jax.ShapeDtypeStruct((HIDDEN, 1), jnp.float32),
            jax.ShapeDtypeStruct((HIDDEN, 1), jnp.float32),
        ],
    )(hT, aggT, w1, b1.reshape(-1, 1), w2, b2.reshape(-1, 1))


def _bn_kernel(z_ref, s_ref, q_ref, g_ref, be_ref, o_ref):
    inv_n = 1.0 / VALID_N
    mean = s_ref[...] * inv_n
    var = q_ref[...] * inv_n - mean * mean
    inv = lax.rsqrt(var + BN_EPS)
    o_ref[...] = _leaky((z_ref[...] - mean) * inv * g_ref[...] + be_ref[...])


def _bn_apply(zT, s, q, gamma, beta, col_blk=2048):
    m, n = zT.shape
    grid = (n // col_blk,)
    return pl.pallas_call(
        _bn_kernel,
        grid=grid,
        in_specs=[
            pl.BlockSpec((m, col_blk), lambda i: (0, i)),
            pl.BlockSpec((m, 1), lambda i: (0, 0)),
            pl.BlockSpec((m, 1), lambda i: (0, 0)),
            pl.BlockSpec((m, 1), lambda i: (0, 0)),
            pl.BlockSpec((m, 1), lambda i: (0, 0)),
        ],
        out_specs=pl.BlockSpec((m, col_blk), lambda i: (0, i)),
        out_shape=jax.ShapeDtypeStruct((m, n), jnp.float32),
    )(zT, s, q, gamma.reshape(-1, 1), beta.reshape(-1, 1))


def _pool_bn_kernel(z_ref, s_ref, q_ref, g_ref, be_ref, o_ref):
    inv_n = 1.0 / VALID_N
    mean = s_ref[...] * inv_n
    var = q_ref[...] * inv_n - mean * mean
    inv = lax.rsqrt(var + BN_EPS)
    h = _leaky((z_ref[...] - mean) * inv * g_ref[...] + be_ref[...])
    hm = jnp.where(_valid_mask(h.shape, 0), h, 0.0)
    p = jnp.sum(hm.reshape(HIDDEN, B, N_PADDED), axis=2) * (1.0 / N_PAD)
    ss = jnp.sum(p * p, axis=0, keepdims=True)
    pn = p / jnp.maximum(jnp.sqrt(ss), 1e-12)
    o_ref[...] = pn.T


def _pool_bn(zT, st, q, gamma, beta):
    return pl.pallas_call(
        _pool_bn_kernel,
        out_shape=jax.ShapeDtypeStruct((B, HIDDEN), jnp.float32),
    )(zT, st, q, gamma.reshape(-1, 1), beta.reshape(-1, 1))


_GATHER_DNUMS = lax.GatherDimensionNumbers(
    offset_dims=(), collapsed_slice_dims=(0,), start_index_map=(0,))


def _lane_pick(vec, idx):
    return lax.gather(vec, idx[:, None], _GATHER_DNUMS, slice_sizes=(1,),
                      mode=lax.GatherScatterMode.PROMISE_IN_BOUNDS)


def _sc_gine_body(h_hbm, e_hbm, ei_hbm, out_hbm,
                  h_v, agg_v, e_v0, src_v0, dst_v0, e_v1, src_v1, dst_v1,
                  e_v2, src_v2, dst_v2, sem0, sem1, sem2):
    c = lax.axis_index("c")
    s = lax.axis_index("s")
    f0 = s * FB
    iota = lax.iota(jnp.int32, 16)
    zeros16 = jnp.zeros((16,), jnp.float32)
    n_steps = (B // 2) * N_CHUNKS

    def chunk_copies(t, bufs):
        e_v, src_v, dst_v, sem = bufs
        gi = t // N_CHUNKS
        ci = t % N_CHUNKS
        g = 2 * gi + c
        eoff = ci * E_CHUNK
        gbase = g * 2 * E_PAD
        return (
            pltpu.make_async_copy(
                ei_hbm.at[pl.ds(gbase + eoff, E_CHUNK)], src_v, sem),
            pltpu.make_async_copy(
                ei_hbm.at[pl.ds(gbase + E_PAD + eoff, E_CHUNK)], dst_v, sem),
            pltpu.make_async_copy(
                e_hbm.at[pl.ds(f0, FB), pl.ds(g * E_PAD + eoff, E_CHUNK)],
                e_v.at[:, pl.ds(0, E_CHUNK)], sem),
        )

    def process_chunk(bufs):
        e_v, src_v, dst_v, _ = bufs

        @plsc.parallel_loop(0, N_GROUPS, unroll=2)
        def group_body(k):
            base = k * 16
            bvec = jnp.full((16,), base, jnp.int32)
            ivec = bvec + iota
            src16 = plsc.load_gather(src_v, [ivec])
            dst16 = plsc.load_gather(dst_v, [ivec])
            for j in range(16):
                jj = jnp.full((16,), j, jnp.int32)
                sb = _lane_pick(src16, jj)
                db = _lane_pick(dst16, jj)
                pv = bvec + j
                erow = plsc.load_gather(e_v, [iota, pv])
                hrow = plsc.load_gather(h_v, [iota, sb])
                m = jnp.maximum(hrow + erow, 0.0)
                plsc.addupdate_scatter(agg_v, [iota, db], m)

    bufs = ((e_v0, src_v0, dst_v0, sem0),
            (e_v1, src_v1, dst_v1, sem1),
            (e_v2, src_v2, dst_v2, sem2))

    for cp in chunk_copies(0, bufs[0]):
        cp.start()
    for cp in chunk_copies(1, bufs[1]):
        cp.start()

    def step_body(t, carry):
        gi = t // N_CHUNKS
        ci = t % N_CHUNKS
        g = 2 * gi + c
        n0 = g * N_PADDED

        @pl.when(ci == 0)
        def _():
            pltpu.sync_copy(h_hbm.at[pl.ds(f0, FB), pl.ds(n0, N_PADDED)],
                            h_v.at[:, pl.ds(0, N_PADDED)])

            @plsc.parallel_loop(0, N_PADDED // 8, unroll=2)
            def zero_body(i):
                base = i * 8
                for u in range(8):
                    plsc.store_scatter(
                        agg_v,
                        [iota, jnp.full((16,), base, jnp.int32) + u],
                        zeros16)

        t2 = jnp.minimum(t + 2, n_steps - 1)

        for r in range(3):
            @pl.when(t % 3 == r)
            def _(r=r):
                for cp in chunk_copies(t, bufs[r]):
                    cp.wait()

                @pl.when(t + 2 < n_steps)
                def _():
                    for cp in chunk_copies(t2, bufs[(r + 2) % 3]):
                        cp.start()

                process_chunk(bufs[r])

        @pl.when(ci == N_CHUNKS - 1)
        def _():
            pltpu.sync_copy(agg_v.at[:, pl.ds(0, N_PADDED)],
                            out_hbm.at[pl.ds(f0, FB), pl.ds(n0, N_PADDED)])
        return carry

    lax.fori_loop(0, n_steps, step_body, 0)


def _sc_gine(hT, eT, ei_flat):
    mesh = plsc.VectorSubcoreMesh(core_axis_name="c", subcore_axis_name="s")
    return pl.kernel(
        _sc_gine_body,
        out_type=jax.ShapeDtypeStruct((HIDDEN, TOTAL_N), jnp.float32),
        mesh=mesh,
        compiler_params=pltpu.CompilerParams(
            use_tc_tiling_on_sc=False, needs_layout_passes=False),
        scratch_types=[
            pltpu.VMEM((FB, N_PADDED + 1), jnp.float32),
            pltpu.VMEM((FB, N_PADDED + 1), jnp.float32),
            pltpu.VMEM((FB, E_CHUNK + 1), jnp.float32),
            pltpu.VMEM((E_CHUNK,), jnp.int32),
            pltpu.VMEM((E_CHUNK,), jnp.int32),
            pltpu.VMEM((FB, E_CHUNK + 1), jnp.float32),
            pltpu.VMEM((E_CHUNK,), jnp.int32),
            pltpu.VMEM((E_CHUNK,), jnp.int32),
            pltpu.VMEM((FB, E_CHUNK + 1), jnp.float32),
            pltpu.VMEM((E_CHUNK,), jnp.int32),
            pltpu.VMEM((E_CHUNK,), jnp.int32),
            pltpu.SemaphoreType.DMA,
            pltpu.SemaphoreType.DMA,
            pltpu.SemaphoreType.DMA,
        ],
    )(hT, eT, ei_flat)


def kernel(x, edge_index, edge_feature, lens, params):
    del lens
    ef = edge_feature.reshape(TOTAL_E, EDGE_DIM)
    ei_flat = edge_index.reshape(B * 2 * E_PAD).astype(jnp.int32)

    hT = _pre_h(x, params["pre_mp_W"], params["pre_mp_b"])
    eT = _pre_e(ef, params["pre_edge_W"], params["pre_edge_b"])

    for l in range(2):
        lp = params["layers"][l]
        aggT = _sc_gine(hT, eT, ei_flat)
        zT, ssum, sq = _layer_mm(hT, aggT, lp["lin1_W"], lp["lin1_b"],
                                 lp["lin2_W"], lp["lin2_b"])
        if l == 0:
            hT = _bn_apply(zT, ssum, sq, lp["bn_gamma"], lp["bn_beta"])
        else:
            return _pool_bn(zT, ssum, sq, lp["bn_gamma"], lp["bn_beta"])


# speedup vs baseline: 1.1517x; 1.0049x over previous
import jax
import jax.numpy as jnp
from jax import lax
from jax.experimental import pallas as pl
from jax.experimental.pallas import tpu as pltpu
from jax.experimental.pallas import tpu_sc as plsc

B = 10
N_PAD = 1000
N_PADDED = 1024
E_PAD = 16000
NODE_DIM = 256
EDGE_DIM = 16
HIDDEN = 256
LEAKY = 0.01
BN_EPS = 1e-5

TOTAL_N = B * N_PADDED
VALID_N = B * N_PAD
TOTAL_E = B * E_PAD
FB = 16
E_CHUNK = 640
N_CHUNKS = E_PAD // E_CHUNK
N_GROUPS = E_CHUNK // 16


def _leaky(v):
    return jnp.where(v > 0, v, LEAKY * v)


def _valid_mask(shape, col0):
    col = col0 + lax.broadcasted_iota(jnp.int32, shape, 1)
    return (col % N_PADDED) < N_PAD


def _pre_h_kernel(x_ref, w_ref, b_ref, o_ref):
    z = (
        lax.dot_general(w_ref[...], x_ref[0], (((1,), (1,)), ((), ())),
                        preferred_element_type=jnp.float32)
        + b_ref[...]
    )
    o_ref[:, : N_PAD] = z
    o_ref[:, N_PAD:] = jnp.zeros((HIDDEN, N_PADDED - N_PAD), jnp.float32)


def _pre_h(x, w, b):
    return pl.pallas_call(
        _pre_h_kernel,
        grid=(B,),
        in_specs=[
            pl.BlockSpec((1, N_PAD, NODE_DIM), lambda g: (g, 0, 0)),
            pl.BlockSpec((HIDDEN, NODE_DIM), lambda g: (0, 0)),
            pl.BlockSpec((HIDDEN, 1), lambda g: (0, 0)),
        ],
        out_specs=pl.BlockSpec((HIDDEN, N_PADDED), lambda g: (0, g)),
        out_shape=jax.ShapeDtypeStruct((HIDDEN, TOTAL_N), jnp.float32),
    )(x, w, b.reshape(-1, 1))


def _pre_e_kernel(ef_ref, w_ref, b_ref, o_ref):
    o_ref[...] = (
        lax.dot_general(w_ref[...], ef_ref[...], (((1,), (1,)), ((), ())),
                        preferred_element_type=jnp.float32)
        + b_ref[...]
    )


def _pre_e(ef, w, b, col_blk=6400):
    n = ef.shape[0]
    return pl.pallas_call(
        _pre_e_kernel,
        grid=(n // col_blk,),
        in_specs=[
            pl.BlockSpec((col_blk, EDGE_DIM), lambda i: (i, 0)),
            pl.BlockSpec((HIDDEN, EDGE_DIM), lambda i: (0, 0)),
            pl.BlockSpec((HIDDEN, 1), lambda i: (0, 0)),
        ],
        out_specs=pl.BlockSpec((HIDDEN, col_blk), lambda i: (0, i)),
        out_shape=jax.ShapeDtypeStruct((HIDDEN, n), jnp.float32),
    )(ef, w, b.reshape(-1, 1))


def _layer_mm_kernel(h_ref, agg_ref, w1_ref, b1_ref, w2_ref, b2_ref,
                     z_ref, s_ref, q_ref):
    u = h_ref[...] + agg_ref[...]
    z1 = _leaky(
        jnp.dot(w1_ref[...], u, preferred_element_type=jnp.float32)
        + b1_ref[...]
    )
    z2 = (
        jnp.dot(w2_ref[...], z1, preferred_element_type=jnp.float32)
        + b2_ref[...]
    )
    z_ref[...] = z2

    @pl.when(pl.program_id(0) == 0)
    def _():
        s_ref[...] = jnp.zeros_like(s_ref)
        q_ref[...] = jnp.zeros_like(q_ref)

    col_blk = z2.shape[1]
    mask = _valid_mask(z2.shape, pl.program_id(0) * col_blk)
    zm = jnp.where(mask, z2, 0.0)
    s_ref[...] += jnp.sum(zm, axis=1, keepdims=True)
    q_ref[...] += jnp.sum(zm * z2, axis=1, keepdims=True)


def _layer_mm(hT, aggT, w1, b1, w2, b2, col_blk=2048):
    m, n = hT.shape
    grid = (n // col_blk,)
    return pl.pallas_call(
        _layer_mm_kernel,
        grid=grid,
        in_specs=[
            pl.BlockSpec((m, col_blk), lambda i: (0, i)),
            pl.BlockSpec((m, col_blk), lambda i: (0, i)),
            pl.BlockSpec((HIDDEN, m), lambda i: (0, 0)),
            pl.BlockSpec((HIDDEN, 1), lambda i: (0, 0)),
            pl.BlockSpec((HIDDEN, HIDDEN), lambda i: (0, 0)),
            pl.BlockSpec((HIDDEN, 1), lambda i: (0, 0)),
        ],
        out_specs=[
            pl.BlockSpec((HIDDEN, col_blk), lambda i: (0, i)),
            pl.BlockSpec((HIDDEN, 1), lambda i: (0, 0)),
            pl.BlockSpec((HIDDEN, 1), lambda i: (0, 0)),
        ],
        out_shape=[
            jax.ShapeDtypeStruct((HIDDEN, n), jnp.float32),
            jax.ShapeDtypeStruct((HIDDEN, 1), jnp.float32),
            jax.ShapeDtypeStruct((HIDDEN, 1), jnp.float32),
        ],
    )(hT, aggT, w1, b1.reshape(-1, 1), w2, b2.reshape(-1, 1))


def _bn_kernel(z_ref, s_ref, q_ref, g_ref, be_ref, o_ref):
    inv_n = 1.0 / VALID_N
    mean = s_ref[...] * inv_n
    var = q_ref[...] * inv_n - mean * mean
    inv = lax.rsqrt(var + BN_EPS)
    o_ref[...] = _leaky((z_ref[...] - mean) * inv * g_ref[...] + be_ref[...])


def _bn_apply(zT, s, q, gamma, beta, col_blk=2048):
    m, n = zT.shape
    grid = (n // col_blk,)
    return pl.pallas_call(
        _bn_kernel,
        grid=grid,
        in_specs=[
            pl.BlockSpec((m, col_blk), lambda i: (0, i)),
            pl.BlockSpec((m, 1), lambda i: (0, 0)),
            pl.BlockSpec((m, 1), lambda i: (0, 0)),
            pl.BlockSpec((m, 1), lambda i: (0, 0)),
            pl.BlockSpec((m, 1), lambda i: (0, 0)),
        ],
        out_specs=pl.BlockSpec((m, col_blk), lambda i: (0, i)),
        out_shape=jax.ShapeDtypeStruct((m, n), jnp.float32),
    )(zT, s, q, gamma.reshape(-1, 1), beta.reshape(-1, 1))


def _pool_bn_kernel(z_ref, s_ref, q_ref, g_ref, be_ref, o_ref):
    inv_n = 1.0 / VALID_N
    mean = s_ref[...] * inv_n
    var = q_ref[...] * inv_n - mean * mean
    inv = lax.rsqrt(var + BN_EPS)
    h = _leaky((z_ref[...] - mean) * inv * g_ref[...] + be_ref[...])
    hm = jnp.where(_valid_mask(h.shape, 0), h, 0.0)
    p = jnp.sum(hm.reshape(HIDDEN, B, N_PADDED), axis=2) * (1.0 / N_PAD)
    ss = jnp.sum(p * p, axis=0, keepdims=True)
    pn = p / jnp.maximum(jnp.sqrt(ss), 1e-12)
    o_ref[...] = pn.T


def _pool_bn(zT, st, q, gamma, beta):
    return pl.pallas_call(
        _pool_bn_kernel,
        out_shape=jax.ShapeDtypeStruct((B, HIDDEN), jnp.float32),
    )(zT, st, q, gamma.reshape(-1, 1), beta.reshape(-1, 1))


_GATHER_DNUMS = lax.GatherDimensionNumbers(
    offset_dims=(), collapsed_slice_dims=(0,), start_index_map=(0,))


def _lane_pick(vec, idx):
    return lax.gather(vec, idx[:, None], _GATHER_DNUMS, slice_sizes=(1,),
                      mode=lax.GatherScatterMode.PROMISE_IN_BOUNDS)


def _sc_gine_body(h_hbm, e_hbm, ei_hbm, out_hbm,
                  h_v, agg_v, e_v0, src_v0, dst_v0, e_v1, src_v1, dst_v1,
                  sem0, sem1):
    c = lax.axis_index("c")
    s = lax.axis_index("s")
    f0 = s * FB
    iota = lax.iota(jnp.int32, 16)
    zeros16 = jnp.zeros((16,), jnp.float32)
    n_steps = (B // 2) * N_CHUNKS

    def chunk_copies(t, bufs):
        e_v, src_v, dst_v, sem = bufs
        gi = t // N_CHUNKS
        ci = t % N_CHUNKS
        g = 2 * gi + c
        eoff = ci * E_CHUNK
        gbase = g * 2 * E_PAD
        return (
            pltpu.make_async_copy(
                ei_hbm.at[pl.ds(gbase + eoff, E_CHUNK)], src_v, sem),
            pltpu.make_async_copy(
                ei_hbm.at[pl.ds(gbase + E_PAD + eoff, E_CHUNK)], dst_v, sem),
            pltpu.make_async_copy(
                e_hbm.at[pl.ds(f0, FB), pl.ds(g * E_PAD + eoff, E_CHUNK)],
                e_v.at[:, pl.ds(0, E_CHUNK)], sem),
        )

    def process_chunk(bufs):
        e_v, src_v, dst_v, _ = bufs

        @plsc.parallel_loop(0, N_GROUPS, unroll=2)
        def group_body(k):
            base = k * 16
            bvec = jnp.full((16,), base, jnp.int32)
            ivec = bvec + iota
            src16 = plsc.load_gather(src_v, [ivec])
            dst16 = plsc.load_gather(dst_v, [ivec])
            for j in range(16):
                jj = jnp.full((16,), j, jnp.int32)
                sb = _lane_pick(src16, jj)
                db = _lane_pick(dst16, jj)
                pv = bvec + j
                erow = plsc.load_gather(e_v, [iota, pv])
                hrow = plsc.load_gather(h_v, [iota, sb])
                m = jnp.maximum(hrow + erow, 0.0)
                plsc.addupdate_scatter(agg_v, [iota, db], m)

    bufs = ((e_v0, src_v0, dst_v0, sem0),
            (e_v1, src_v1, dst_v1, sem1))

    for cp in chunk_copies(0, bufs[0]):
        cp.start()

    def step_body(t, carry):
        gi = t // N_CHUNKS
        ci = t % N_CHUNKS
        g = 2 * gi + c
        n0 = g * N_PADDED

        @pl.when(ci == 0)
        def _():
            pltpu.sync_copy(h_hbm.at[pl.ds(f0, FB), pl.ds(n0, N_PADDED)],
                            h_v.at[:, pl.ds(0, N_PADDED)])

            @plsc.parallel_loop(0, N_PADDED // 8, unroll=2)
            def zero_body(i):
                base = i * 8
                for u in range(8):
                    plsc.store_scatter(
                        agg_v,
                        [iota, jnp.full((16,), base, jnp.int32) + u],
                        zeros16)

        t2 = jnp.minimum(t + 1, n_steps - 1)

        for r in range(2):
            @pl.when(t % 2 == r)
            def _(r=r):
                for cp in chunk_copies(t, bufs[r]):
                    cp.wait()

                @pl.when(t + 1 < n_steps)
                def _():
                    for cp in chunk_copies(t2, bufs[1 - r]):
                        cp.start()

                process_chunk(bufs[r])

        @pl.when(ci == N_CHUNKS - 1)
        def _():
            pltpu.sync_copy(agg_v.at[:, pl.ds(0, N_PADDED)],
                            out_hbm.at[pl.ds(f0, FB), pl.ds(n0, N_PADDED)])
        return carry

    lax.fori_loop(0, n_steps, step_body, 0)


def _sc_gine(hT, eT, ei_flat):
    mesh = plsc.VectorSubcoreMesh(core_axis_name="c", subcore_axis_name="s")
    return pl.kernel(
        _sc_gine_body,
        out_type=jax.ShapeDtypeStruct((HIDDEN, TOTAL_N), jnp.float32),
        mesh=mesh,
        compiler_params=pltpu.CompilerParams(
            use_tc_tiling_on_sc=False, needs_layout_passes=False),
        scratch_types=[
            pltpu.VMEM((FB, N_PADDED + 1), jnp.float32),
            pltpu.VMEM((FB, N_PADDED + 1), jnp.float32),
            pltpu.VMEM((FB, E_CHUNK + 1), jnp.float32),
            pltpu.VMEM((E_CHUNK,), jnp.int32),
            pltpu.VMEM((E_CHUNK,), jnp.int32),
            pltpu.VMEM((FB, E_CHUNK + 1), jnp.float32),
            pltpu.VMEM((E_CHUNK,), jnp.int32),
            pltpu.VMEM((E_CHUNK,), jnp.int32),
            pltpu.SemaphoreType.DMA,
            pltpu.SemaphoreType.DMA,
        ],
    )(hT, eT, ei_flat)


def kernel(x, edge_index, edge_feature, lens, params):
    del lens
    ef = edge_feature.reshape(TOTAL_E, EDGE_DIM)
    ei_flat = edge_index.reshape(B * 2 * E_PAD).astype(jnp.int32)

    hT = _pre_h(x, params["pre_mp_W"], params["pre_mp_b"])
    eT = _pre_e(ef, params["pre_edge_W"], params["pre_edge_b"])

    for l in range(2):
        lp = params["layers"][l]
        aggT = _sc_gine(hT, eT, ei_flat)
        zT, ssum, sq = _layer_mm(hT, aggT, lp["lin1_W"], lp["lin1_b"],
                                 lp["lin2_W"], lp["lin2_b"])
        if l == 0:
            hT = _bn_apply(zT, ssum, sq, lp["bn_gamma"], lp["bn_beta"])
        else:
            return _pool_bn(zT, ssum, sq, lp["bn_gamma"], lp["bn_beta"])
